# Initial kernel scaffold; baseline (speedup 1.0000x reference)
import jax
import jax.numpy as jnp
from jax.experimental import pallas as pl


def kernel(x, edge_index, edge_attr, graph_feat, W_node, b_node, W_edge, b_edge, Wmi, bmi, Wmo, bmo, ln_ng, ln_nb, ln_eg, ln_eb, W_gate, b_gate, W_gap, b_gap, W_gf, b_gf, W_f1, b_f1, W_f2, b_f2):
    raise NotImplementedError("write your pallas kernel here")


# speedup vs baseline: 1.4845x; 1.4845x over previous
import functools

import jax
import jax.numpy as jnp
from jax import lax
from jax.experimental import pallas as pl
from jax.experimental.pallas import tpu as pltpu
from jax.experimental.pallas import tpu_sc as plsc

H = 32
K = H // 2
HH = H * K
NC = 2
NS = 16
NW = NC * NS
EPS = 1e-5


def _b16(v):
    return v.astype(jnp.bfloat16)


def _b16f(v):
    return v.astype(jnp.bfloat16).astype(jnp.float32)


def _node_init_body(x_r, wn_r, bn_r, g_r, b_r, h_r, hl_r):
    h = jnp.maximum(
        jnp.dot(_b16(x_r[...]), _b16(wn_r[...]),
                preferred_element_type=jnp.float32)
        + bn_r[...], 0.0)
    h_r[...] = h
    mu = jnp.mean(h, axis=1, keepdims=True)
    d = h - mu
    var = jnp.mean(d * d, axis=1, keepdims=True)
    hl_r[...] = d * lax.rsqrt(var + EPS) * g_r[...] + b_r[...]


def _edge_body(ea_r, hd_r, hs_r, we_r, be_r, wmi_r, bmi_r, wmo_r, bmo_r,
               eg_r, eb_r, s_r, mi_r, mo_r):
    e = jnp.maximum(_b16f(ea_r[...]) * _b16f(we_r[...]) + be_r[...], 0.0)
    mu = jnp.mean(e, axis=1, keepdims=True)
    d = e - mu
    var = jnp.mean(d * d, axis=1, keepdims=True)
    eh = _b16(d * lax.rsqrt(var + EPS) * eg_r[...] + eb_r[...])

    ti = jnp.maximum(
        jnp.dot(eh, _b16(wmi_r[...]), preferred_element_type=jnp.float32)
        + bmi_r[...], 0.0)
    hdt = jnp.concatenate([_b16f(hd_r[...])] * K, axis=1)
    mi_r[...] = jnp.dot(_b16f(ti) * hdt, s_r[...],
                        precision=lax.Precision.HIGHEST,
                        preferred_element_type=jnp.float32)

    to = jnp.maximum(
        jnp.dot(eh, _b16(wmo_r[...]), preferred_element_type=jnp.float32)
        + bmo_r[...], 0.0)
    hst = jnp.concatenate([_b16f(hs_r[...])] * K, axis=1)
    mo_r[...] = jnp.dot(_b16f(to) * hst, s_r[...],
                        precision=lax.Precision.HIGHEST,
                        preferred_element_type=jnp.float32)


def _update_body(h_r, ai_r, ao_r, g_r, b_r, hn_r, hl_r):
    m = jnp.concatenate([ai_r[0] + ai_r[1], ao_r[0] + ao_r[1]], axis=1)
    hn = jnp.maximum(h_r[...] + m, 0.0)
    hn_r[...] = hn
    mu = jnp.mean(hn, axis=1, keepdims=True)
    d = hn - mu
    var = jnp.mean(d * d, axis=1, keepdims=True)
    hl_r[...] = d * lax.rsqrt(var + EPS) * g_r[...] + b_r[...]


def _make_head_body(n_real):
    def _head_body(h_r, ai_r, ao_r, wg_r, bg_r, wgap_r, bgap_r, wgf_r, bgf_r,
                   gf_r, wf1_r, bf1_r, wf2_r, bf2_r, pred_r):
        m = jnp.concatenate([ai_r[0] + ai_r[1], ao_r[0] + ao_r[1]], axis=1)
        h2 = jnp.maximum(h_r[...] + m, 0.0)
        rows = lax.broadcasted_iota(jnp.int32, h2.shape, 0)
        valid = rows < n_real
        h2 = jnp.where(valid, h2, 0.0)
        z = jnp.sum(_b16f(h2) * _b16f(wg_r[...]), axis=1, keepdims=True) + bg_r[...]
        z = jnp.where(valid[:, :1], z, -1e30)
        zm = jnp.max(z, axis=0, keepdims=True)
        p = jnp.exp(z - zm)
        p = jnp.where(valid[:, :1], p, 0.0)
        s = jnp.sum(p, axis=0, keepdims=True)
        gate = p / s
        pooled = jnp.sum(gate * h2, axis=0, keepdims=True)
        h0 = jnp.maximum(
            jnp.dot(_b16(pooled), _b16(wgap_r[...]),
                    preferred_element_type=jnp.float32)
            + bgap_r[...], 0.0)
        h1 = jnp.maximum(_b16f(gf_r[...]) * _b16f(wgf_r[...]) + bgf_r[...], 0.0)
        hc = jnp.concatenate([h0, h1], axis=1)
        z1 = jnp.maximum(
            jnp.dot(_b16(hc), _b16(wf1_r[...]),
                    preferred_element_type=jnp.float32)
            + bf1_r[...], 0.0)
        pred_r[...] = (jnp.sum(_b16f(z1) * _b16f(wf2_r[...]), axis=1,
                               keepdims=True) + bf2_r[...])
    return _head_body


def _make_gather(npad, epad):
    per_w = epad // NW
    rows_per_w = per_w // 128
    n_chunks = per_w // 1024
    mesh = plsc.VectorSubcoreMesh(core_axis_name="c", subcore_axis_name="s")

    @functools.partial(
        pl.kernel, mesh=mesh,
        compiler_params=pltpu.CompilerParams(use_tc_tiling_on_sc=False),
        out_type=(jax.ShapeDtypeStruct((epad, H), jnp.float32),
                  jax.ShapeDtypeStruct((epad, H), jnp.float32)),
        scratch_types=[
            pltpu.VMEM((8, 128), jnp.int32),
            pltpu.VMEM((8, 128), jnp.int32),
            pltpu.VMEM((1024, H), jnp.float32),
            pltpu.VMEM((1024, H), jnp.float32),
            pltpu.SemaphoreType.DMA,
            pltpu.SemaphoreType.DMA,
        ],
    )
    def gather(hln_hbm, dst_hbm, src_hbm, hd_hbm, hs_hbm,
               idx_d, idx_s, rows_d, rows_s, sem_d, sem_s):
        wid = lax.axis_index("s") * NC + lax.axis_index("c")
        for j in range(n_chunks):
            erow = wid * rows_per_w + j * 8
            ebase = wid * per_w + j * 1024
            pltpu.sync_copy(dst_hbm.at[pl.ds(erow, 8)], idx_d)
            pltpu.sync_copy(src_hbm.at[pl.ds(erow, 8)], idx_s)
            cps = []
            for b in range(8):
                cps.append(pltpu.make_async_copy(
                    hln_hbm.at[idx_d.at[b]],
                    rows_d.at[pl.ds(b * 128, 128)], sem_d))
                cps.append(pltpu.make_async_copy(
                    hln_hbm.at[idx_s.at[b]],
                    rows_s.at[pl.ds(b * 128, 128)], sem_s))
            for cp in cps:
                cp.start()
            for cp in cps:
                cp.wait()
            pltpu.sync_copy(rows_d, hd_hbm.at[pl.ds(ebase, 1024)])
            pltpu.sync_copy(rows_s, hs_hbm.at[pl.ds(ebase, 1024)])

    return gather


def _make_scatter(npad, epad):
    per_w = epad // NW
    rows_per_w = per_w // 128
    n_chunks = per_w // 1024
    npt = npad // NS
    mesh = plsc.VectorSubcoreMesh(core_axis_name="c", subcore_axis_name="s")

    @functools.partial(
        pl.kernel, mesh=mesh,
        compiler_params=pltpu.CompilerParams(use_tc_tiling_on_sc=False),
        out_type=(jax.ShapeDtypeStruct((NC, npad, K), jnp.float32),
                  jax.ShapeDtypeStruct((NC, npad, K), jnp.float32)),
        scratch_types=[
            pltpu.VMEM((8, 128), jnp.int32),
            pltpu.VMEM((8, 128), jnp.int32),
            pltpu.VMEM((1024, K), jnp.float32),
            pltpu.VMEM((1024, K), jnp.float32),
            pltpu.VMEM((npad // NS, K), jnp.float32),
            pltpu.VMEM_SHARED((npad, K), jnp.float32),
            pltpu.VMEM_SHARED((npad, K), jnp.float32),
        ],
    )
    def scatter(mi_hbm, mo_hbm, dst_hbm, src_hbm, zero_hbm,
                acc_i_out, acc_o_out,
                idx_d, idx_s, mi_v, mo_v, stage_v, acc_i_sh, acc_o_sh):
        c = lax.axis_index("c")
        s = lax.axis_index("s")
        wid = s * NC + c
        pltpu.sync_copy(zero_hbm.at[pl.ds(s * npt, npt)], stage_v)
        pltpu.sync_copy(stage_v, acc_i_sh.at[pl.ds(s * npt, npt)])
        pltpu.sync_copy(stage_v, acc_o_sh.at[pl.ds(s * npt, npt)])
        plsc.subcore_barrier()
        for j in range(n_chunks):
            erow = wid * rows_per_w + j * 8
            ebase = wid * per_w + j * 1024
            pltpu.sync_copy(dst_hbm.at[pl.ds(erow, 8)], idx_d)
            pltpu.sync_copy(src_hbm.at[pl.ds(erow, 8)], idx_s)
            pltpu.sync_copy(mi_hbm.at[pl.ds(ebase, 1024)], mi_v)
            pltpu.sync_copy(mo_hbm.at[pl.ds(ebase, 1024)], mo_v)
            for b in range(8):
                pltpu.sync_copy(mi_v.at[pl.ds(b * 128, 128)],
                                acc_i_sh.at[idx_d.at[b]], add=True)
                pltpu.sync_copy(mo_v.at[pl.ds(b * 128, 128)],
                                acc_o_sh.at[idx_s.at[b]], add=True)
        plsc.subcore_barrier()
        pltpu.sync_copy(acc_i_sh.at[pl.ds(s * npt, npt)], stage_v)
        pltpu.sync_copy(stage_v, acc_i_out.at[c, pl.ds(s * npt, npt)])
        pltpu.sync_copy(acc_o_sh.at[pl.ds(s * npt, npt)], stage_v)
        pltpu.sync_copy(stage_v, acc_o_out.at[c, pl.ds(s * npt, npt)])

    return scatter


def kernel(x, edge_index, edge_attr, graph_feat, W_node, b_node, W_edge,
           b_edge, Wmi, bmi, Wmo, bmo, ln_ng, ln_nb, ln_eg, ln_eb,
           W_gate, b_gate, W_gap, b_gap, W_gf, b_gf, W_f1, b_f1, W_f2, b_f2):
    f32 = jnp.float32
    N = x.shape[0]
    E = edge_index.shape[1]
    L = Wmi.shape[0]
    npad = ((N + 127) // 128) * 128
    epad = ((E + NW * 1024 - 1) // (NW * 1024)) * (NW * 1024)

    src = jnp.pad(edge_index[0], (0, epad - E), constant_values=N)
    dst = jnp.pad(edge_index[1], (0, epad - E), constant_values=N)
    src2d = src.reshape(epad // 128, 128)
    dst2d = dst.reshape(epad // 128, 128)
    ea = jnp.pad(edge_attr, ((0, epad - E), (0, 0)))
    xp = jnp.pad(x, ((0, npad - N), (0, 6)))
    wn = jnp.pad(W_node, ((0, 6), (0, 0)))
    zeros_nk = jnp.zeros((npad, K), f32)

    sel = (jnp.arange(HH)[:, None] // H == jnp.arange(K)[None, :]).astype(f32)

    r1 = lambda a: a.reshape(1, -1)

    h, h_ln = pl.pallas_call(
        _node_init_body,
        out_shape=(jax.ShapeDtypeStruct((npad, H), f32),
                   jax.ShapeDtypeStruct((npad, H), f32)),
    )(xp, wn, r1(b_node), r1(ln_ng[0]), r1(ln_nb[0]))

    gather = _make_gather(npad, epad)
    scatter = _make_scatter(npad, epad)

    B = 2560
    grid = epad // B
    edge_call = pl.pallas_call(
        _edge_body,
        grid=(grid,),
        in_specs=[
            pl.BlockSpec((B, 1), lambda i: (i, 0)),
            pl.BlockSpec((B, H), lambda i: (i, 0)),
            pl.BlockSpec((B, H), lambda i: (i, 0)),
            pl.BlockSpec((1, H), lambda i: (0, 0)),
            pl.BlockSpec((1, H), lambda i: (0, 0)),
            pl.BlockSpec((H, HH), lambda i: (0, 0)),
            pl.BlockSpec((1, HH), lambda i: (0, 0)),
            pl.BlockSpec((H, HH), lambda i: (0, 0)),
            pl.BlockSpec((1, HH), lambda i: (0, 0)),
            pl.BlockSpec((1, H), lambda i: (0, 0)),
            pl.BlockSpec((1, H), lambda i: (0, 0)),
            pl.BlockSpec((HH, K), lambda i: (0, 0)),
        ],
        out_specs=[pl.BlockSpec((B, K), lambda i: (i, 0)),
                   pl.BlockSpec((B, K), lambda i: (i, 0))],
        out_shape=[jax.ShapeDtypeStruct((epad, K), f32),
                   jax.ShapeDtypeStruct((epad, K), f32)],
    )

    for l in range(L):
        hd, hs = gather(h_ln, dst2d, src2d)
        m_in, m_out = edge_call(ea, hd, hs, r1(W_edge), r1(b_edge),
                                Wmi[l], r1(bmi[l]), Wmo[l], r1(bmo[l]),
                                r1(ln_eg[l]), r1(ln_eb[l]), sel)
        acc_i, acc_o = scatter(m_in, m_out, dst2d, src2d, zeros_nk)
        if l + 1 < L:
            h, h_ln = pl.pallas_call(
                _update_body,
                out_shape=(jax.ShapeDtypeStruct((npad, H), f32),
                           jax.ShapeDtypeStruct((npad, H), f32)),
            )(h, acc_i, acc_o, r1(ln_ng[l + 1]), r1(ln_nb[l + 1]))

    pred = pl.pallas_call(
        _make_head_body(N),
        out_shape=jax.ShapeDtypeStruct((1, 1), f32),
    )(h, acc_i, acc_o, r1(W_gate[:, 0]), r1(b_gate), W_gap, r1(b_gap),
      r1(W_gf[0]), r1(b_gf), r1(graph_feat), W_f1, r1(b_f1),
      r1(W_f2[:, 0]), r1(b_f2))
    return pred.reshape(1)
